# baseline (speedup 1.0000x reference)
import functools

import numpy as np
import jax
import jax.numpy as jnp
from jax import lax
from jax.experimental import pallas as pl
from jax.experimental.pallas import tpu as pltpu
from jax.experimental.pallas import tpu_sc as plsc

_N_IMG = 4
_A = 20000
_CHUNK = 2496
_LAST = _A - 7 * _CHUNK
_NV = _CHUNK // 16
_NV_LAST = _LAST // 16
_NBINS = 11
_L = 16

_MAGIC = np.int32(0x5F3759DF)


_GATHER_DN = lax.GatherDimensionNumbers(
    offset_dims=(), collapsed_slice_dims=(0,), start_index_map=(0,))


def _perm(v, idx):
    return lax.gather(v, idx[:, None], _GATHER_DN, slice_sizes=(1,),
                      mode=lax.GatherScatterMode.PROMISE_IN_BOUNDS)


def _rsqrt_newton(s):
    r = lax.bitcast_convert_type(
        _MAGIC - lax.shift_right_arithmetic(
            lax.bitcast_convert_type(s, jnp.int32), 1), jnp.float32)
    for _ in range(2):
        r = r * (1.5 - 0.5 * s * r * r)
    return r


def _sc_body(pr_hbm, tr_hbm, pd_hbm, td_hbm, out_hbm,
             bufp, buft, bufpd, buftd, hp, ht, hm, hd,
             row64, shared, t0buf, resbuf):
    cid = lax.axis_index("c")
    sid = lax.axis_index("s")
    im = cid * 2 + sid // 8
    sub = sid % 8
    lane = lax.iota(jnp.int32, _L)

    roff = im * _A + sub * _CHUNK
    doff = 2 * roff
    pltpu.sync_copy(pr_hbm.at[pl.ds(roff, _LAST)], bufp)
    pltpu.sync_copy(tr_hbm.at[pl.ds(roff, _LAST)], buft)
    pltpu.sync_copy(pd_hbm.at[pl.ds(doff, 2 * _LAST)], bufpd)
    pltpu.sync_copy(td_hbm.at[pl.ds(doff, 2 * _LAST)], buftd)

    zero = jnp.zeros((_L,), jnp.float32)
    for b in range(_NBINS):
        hp[pl.ds(b * _L, _L)] = zero
        ht[pl.ds(b * _L, _L)] = zero
        hm[pl.ds(b * _L, _L)] = zero
        hd[pl.ds(b * _L, _L)] = zero

    nv = jnp.where(sub == 7, _NV_LAST, _NV)
    ones = jnp.ones((_L,), jnp.float32)

    def body(j, _):
        p = bufp[pl.ds(j * _L, _L)]
        t = buft[pl.ds(j * _L, _L)]
        dv0 = buftd[pl.ds(j * 2 * _L, _L)] - bufpd[pl.ds(j * 2 * _L, _L)]
        dv1 = buftd[pl.ds(j * 2 * _L + _L, _L)] - bufpd[pl.ds(j * 2 * _L + _L, _L)]
        sq0 = dv0 * dv0
        sq1 = dv1 * dv1
        p0 = sq0 + _perm(sq0, lane ^ 1)
        p1 = sq1 + _perm(sq1, lane ^ 1)
        idxm = (2 * lane) & 15
        s = jnp.where(lane < 8, _perm(p0, idxm), _perm(p1, idxm))
        dist = s * _rsqrt_newton(s)
        sig = 1.0 / (1.0 + jnp.exp(-p))
        lvl_p = ((sig + 0.05) * 10.0).astype(jnp.int32)
        lvl_t = ((t + 0.05) * 10.0).astype(jnp.int32)
        lvl_m = jnp.minimum(lvl_p, lvl_t)
        plsc.addupdate_scatter(hp, [lvl_p * _L + lane], ones)
        plsc.addupdate_scatter(ht, [lvl_t * _L + lane], ones)
        plsc.addupdate_scatter(hm, [lvl_m * _L + lane], ones)
        plsc.addupdate_scatter(hd, [lvl_m * _L + lane], dist)
        return _

    lax.fori_loop(0, nv, body, None)

    for slot, h in enumerate((hp, ht, hm, hd)):
        binvec = jnp.zeros((_L,), jnp.float32)
        for b in range(_NBINS):
            sb = jnp.sum(h[pl.ds(b * _L, _L)])
            binvec = jnp.where(lane == b, sb, binvec)
        row64[pl.ds(slot * _L, _L)] = binvec
    pltpu.sync_copy(row64, shared.at[pl.ds(sid * (4 * _L), 4 * _L)])
    plsc.subcore_barrier()

    @pl.when(sid == 0)
    def _():
        pltpu.sync_copy(shared, t0buf)
        pr_acc = jnp.float32(0.0)
        rc_acc = jnp.float32(0.0)
        de_acc = jnp.float32(0.0)
        for half in range(2):
            hsum = [jnp.zeros((_L,), jnp.float32) for _ in range(4)]
            for w in range(8):
                for slot in range(4):
                    base = (half * 8 + w) * (4 * _L) + slot * _L
                    hsum[slot] = hsum[slot] + t0buf[pl.ds(base, _L)]
            sfx = []
            for slot in range(4):
                h = hsum[slot]
                c = lax.rev(jnp.cumsum(lax.rev(h, (0,))), (0,))
                sfx.append(c - h)
            pos_s, tru_s, tp_s, d_s = sfx
            pr_acc = pr_acc + jnp.sum(tp_s / (pos_s + 1.0))
            rc_acc = rc_acc + jnp.sum(tp_s / (tru_s + 1.0))
            de_acc = de_acc + jnp.sum(d_s / jnp.maximum(tp_s, 1.0))
        scale = jnp.float32(1.0 / (10.0 * _N_IMG))
        res = jnp.where(lane == 0, pr_acc * scale,
                        jnp.where(lane == 1, rc_acc * scale,
                                  jnp.where(lane == 2, de_acc * scale, 0.0)))
        resbuf[...] = res
        pltpu.sync_copy(resbuf, out_hbm.at[cid])


_sc_kernel = functools.partial(
    pl.kernel,
    mesh=plsc.VectorSubcoreMesh(core_axis_name="c", subcore_axis_name="s"),
    out_type=jax.ShapeDtypeStruct((2, _L), jnp.float32),
    compiler_params=pltpu.CompilerParams(needs_layout_passes=False),
    scratch_types=[
        pltpu.VMEM((_LAST,), jnp.float32),
        pltpu.VMEM((_LAST,), jnp.float32),
        pltpu.VMEM((2 * _LAST,), jnp.float32),
        pltpu.VMEM((2 * _LAST,), jnp.float32),
        pltpu.VMEM((_NBINS * _L,), jnp.float32),
        pltpu.VMEM((_NBINS * _L,), jnp.float32),
        pltpu.VMEM((_NBINS * _L,), jnp.float32),
        pltpu.VMEM((_NBINS * _L,), jnp.float32),
        pltpu.VMEM((4 * _L,), jnp.float32),
        pltpu.VMEM_SHARED((16 * 4 * _L,), jnp.float32),
        pltpu.VMEM((16 * 4 * _L,), jnp.float32),
        pltpu.VMEM((_L,), jnp.float32),
    ],
)(_sc_body)


def kernel(pred_rank, pred_disp_vector, target_rank, target_disp_vector, anchors):
    del anchors
    out = _sc_kernel(
        pred_rank.reshape(-1),
        target_rank.reshape(-1),
        pred_disp_vector.reshape(-1),
        target_disp_vector.reshape(-1),
    )
    s = out[0] + out[1]
    return (s[0], s[1], s[2])


# speedup vs baseline: 4.0551x; 4.0551x over previous
import functools

import numpy as np
import jax
import jax.numpy as jnp
from jax import lax
from jax.experimental import pallas as pl
from jax.experimental.pallas import tpu as pltpu
from jax.experimental.pallas import tpu_sc as plsc

_N_IMG = 4
_A = 20000
_TILE = 128
_NTILES = _A // _TILE
_CUT = _NTILES * _TILE
_CHUNK = 2560
_LASTC = _CUT - 7 * _CHUNK
_NV = _CHUNK // 16
_NV_LAST = _LASTC // 16
_NBINS = 11
_L = 16

_MAGIC = np.int32(0x5F3759DF)
_THR = [0.1 * i + 0.05 for i in range(10)]


def _rsqrt_newton(s):
    r = lax.bitcast_convert_type(
        _MAGIC - lax.shift_right_arithmetic(
            lax.bitcast_convert_type(s, jnp.int32), 1), jnp.float32)
    for _ in range(2):
        r = r * (1.5 - 0.5 * s * r * r)
    return r


def _sc_body(pr_hbm, tr_hbm, pdt_hbm, tdt_hbm, out_hbm,
             bufp, buft, bufpx, bufpy, buftx, bufty, hp, ht, hm, hd, row64):
    cid = lax.axis_index("c")
    sid = lax.axis_index("s")
    im = cid * 2 + sid // 8
    sub = sid % 8
    lane = lax.iota(jnp.int32, _L)
    base = sub * _CHUNK

    def stage(n):
        pltpu.sync_copy(pr_hbm.at[im, pl.ds(base, n)], bufp.at[pl.ds(0, n)])
        pltpu.sync_copy(tr_hbm.at[im, pl.ds(base, n)], buft.at[pl.ds(0, n)])
        pltpu.sync_copy(pdt_hbm.at[im, 0, pl.ds(base, n)], bufpx.at[pl.ds(0, n)])
        pltpu.sync_copy(pdt_hbm.at[im, 1, pl.ds(base, n)], bufpy.at[pl.ds(0, n)])
        pltpu.sync_copy(tdt_hbm.at[im, 0, pl.ds(base, n)], buftx.at[pl.ds(0, n)])
        pltpu.sync_copy(tdt_hbm.at[im, 1, pl.ds(base, n)], bufty.at[pl.ds(0, n)])

    @pl.when(sub == 7)
    def _():
        stage(_LASTC)

    @pl.when(sub != 7)
    def _():
        stage(_CHUNK)

    zero = jnp.zeros((_L,), jnp.float32)
    for b in range(_NBINS):
        hp[pl.ds(b * _L, _L)] = zero
        ht[pl.ds(b * _L, _L)] = zero
        hm[pl.ds(b * _L, _L)] = zero
        hd[pl.ds(b * _L, _L)] = zero

    nv = jnp.where(sub == 7, _NV_LAST, _NV)
    ones = jnp.ones((_L,), jnp.float32)

    def body(j, _):
        o = j * _L
        p = bufp[pl.ds(o, _L)]
        t = buft[pl.ds(o, _L)]
        dx = buftx[pl.ds(o, _L)] - bufpx[pl.ds(o, _L)]
        dy = bufty[pl.ds(o, _L)] - bufpy[pl.ds(o, _L)]
        s = dx * dx + dy * dy
        dist = s * _rsqrt_newton(s)
        sig = 1.0 / (1.0 + jnp.exp(-p))
        lvl_p = ((sig + 0.05) * 10.0).astype(jnp.int32)
        lvl_t = ((t + 0.05) * 10.0).astype(jnp.int32)
        lvl_m = jnp.minimum(lvl_p, lvl_t)
        plsc.addupdate_scatter(hp, [lvl_p * _L + lane], ones)
        plsc.addupdate_scatter(ht, [lvl_t * _L + lane], ones)
        plsc.addupdate_scatter(hm, [lvl_m * _L + lane], ones)
        plsc.addupdate_scatter(hd, [lvl_m * _L + lane], dist)
        return _

    lax.fori_loop(0, nv, body, None)

    for slot, h in enumerate((hp, ht, hm, hd)):
        binvec = jnp.zeros((_L,), jnp.float32)
        for b in range(_NBINS):
            sb = jnp.sum(h[pl.ds(b * _L, _L)])
            binvec = jnp.where(lane == b, sb, binvec)
        row64[pl.ds(slot * _L, _L)] = binvec
    row = im * 8 + sub
    pltpu.sync_copy(row64, out_hbm.at[pl.ds(row * (4 * _L), 4 * _L)])


_sc_kernel = functools.partial(
    pl.kernel,
    mesh=plsc.VectorSubcoreMesh(core_axis_name="c", subcore_axis_name="s"),
    out_type=jax.ShapeDtypeStruct((32 * 4 * _L,), jnp.float32),
    compiler_params=pltpu.CompilerParams(needs_layout_passes=False),
    scratch_types=[
        pltpu.VMEM((_CHUNK,), jnp.float32),
        pltpu.VMEM((_CHUNK,), jnp.float32),
        pltpu.VMEM((_CHUNK,), jnp.float32),
        pltpu.VMEM((_CHUNK,), jnp.float32),
        pltpu.VMEM((_CHUNK,), jnp.float32),
        pltpu.VMEM((_CHUNK,), jnp.float32),
        pltpu.VMEM((_NBINS * _L,), jnp.float32),
        pltpu.VMEM((_NBINS * _L,), jnp.float32),
        pltpu.VMEM((_NBINS * _L,), jnp.float32),
        pltpu.VMEM((_NBINS * _L,), jnp.float32),
        pltpu.VMEM((4 * _L,), jnp.float32),
    ],
)(_sc_body)

def _tc_body(s_ref, pr_ref, tr_ref, pdt_ref, tdt_ref, pr_out, rc_out, de_out):
    lane16 = lax.iota(jnp.int32, _L)
    ut = (lax.broadcasted_iota(jnp.int32, (_L, _L), 0)
          > lax.broadcasted_iota(jnp.int32, (_L, _L), 1)).astype(jnp.float32)
    valid = lax.iota(jnp.int32, _TILE) < (_A - _CUT)

    pr_acc = jnp.float32(0.0)
    rc_acc = jnp.float32(0.0)
    de_acc = jnp.float32(0.0)
    for im in range(_N_IMG):
        rowsum = jnp.zeros((4 * _L,), jnp.float32)
        for w in range(8):
            rowsum = rowsum + s_ref[pl.ds((im * 8 + w) * 4 * _L, 4 * _L)]
        p = pr_ref[im]
        t = tr_ref[im]
        sig = jax.nn.sigmoid(p)
        dx = tdt_ref[im, 0] - pdt_ref[im, 0]
        dy = tdt_ref[im, 1] - pdt_ref[im, 1]
        dist = jnp.sqrt(dx * dx + dy * dy)
        sfx = []
        for slot in range(4):
            h = lax.slice(rowsum, (slot * _L,), ((slot + 1) * _L,))
            c = jnp.dot(h.reshape(1, _L), ut, preferred_element_type=jnp.float32)
            tail = jnp.zeros((_L,), jnp.float32)
            for i in range(10):
                pos = jnp.logical_and(valid, sig > _THR[i])
                tru = jnp.logical_and(valid, t > _THR[i])
                if slot == 0:
                    cnt = jnp.sum(pos.astype(jnp.float32))
                elif slot == 1:
                    cnt = jnp.sum(tru.astype(jnp.float32))
                elif slot == 2:
                    cnt = jnp.sum(jnp.logical_and(pos, tru).astype(jnp.float32))
                else:
                    cnt = jnp.sum(jnp.where(jnp.logical_and(pos, tru), dist, 0.0))
                tail = jnp.where(lane16 == i, cnt, tail)
            sfx.append(c.reshape(_L) + tail)
        pos_s, tru_s, tp_s, d_s = sfx
        pr_acc = pr_acc + jnp.sum(tp_s / (pos_s + 1.0))
        rc_acc = rc_acc + jnp.sum(tp_s / (tru_s + 1.0))
        de_acc = de_acc + jnp.sum(d_s / jnp.maximum(tp_s, 1.0))
    scale = jnp.float32(1.0 / (10.0 * _N_IMG))
    pr_out[0, 0] = pr_acc * scale
    rc_out[0, 0] = rc_acc * scale
    de_out[0, 0] = de_acc * scale


def _tc_merge(s, pred_rank, target_rank, pdt, tdt):
    return pl.pallas_call(
        _tc_body,
        grid=(1,),
        out_shape=(
            jax.ShapeDtypeStruct((1, 1), jnp.float32),
            jax.ShapeDtypeStruct((1, 1), jnp.float32),
            jax.ShapeDtypeStruct((1, 1), jnp.float32),
        ),
        in_specs=[
            pl.BlockSpec((32 * 4 * _L,), lambda i: (0,)),
            pl.BlockSpec((_N_IMG, _TILE), lambda i: (0, _NTILES)),
            pl.BlockSpec((_N_IMG, _TILE), lambda i: (0, _NTILES)),
            pl.BlockSpec((_N_IMG, 2, _TILE), lambda i: (0, 0, _NTILES)),
            pl.BlockSpec((_N_IMG, 2, _TILE), lambda i: (0, 0, _NTILES)),
        ],
        out_specs=(
            pl.BlockSpec(memory_space=pltpu.SMEM),
            pl.BlockSpec(memory_space=pltpu.SMEM),
            pl.BlockSpec(memory_space=pltpu.SMEM),
        ),
    )(s, pred_rank, target_rank, pdt, tdt)


def kernel(pred_rank, pred_disp_vector, target_rank, target_disp_vector, anchors):
    del anchors
    pdt = jnp.transpose(pred_disp_vector, (0, 2, 1))
    tdt = jnp.transpose(target_disp_vector, (0, 2, 1))
    s = _sc_kernel(pred_rank, target_rank, pdt, tdt)
    o = _tc_merge(s, pred_rank, target_rank, pdt, tdt)
    return (o[0][0, 0], o[1][0, 0], o[2][0, 0])


# speedup vs baseline: 4.3720x; 1.0781x over previous
import functools

import numpy as np
import jax
import jax.numpy as jnp
from jax import lax
from jax.experimental import pallas as pl
from jax.experimental.pallas import tpu as pltpu
from jax.experimental.pallas import tpu_sc as plsc

_N_IMG = 4
_A = 20000
_TILE = 128
_NTILES = _A // _TILE
_CUT = _NTILES * _TILE
_TAIL = _A - _CUT
_CHUNK = 2560
_LASTC = _CUT - 7 * _CHUNK
_NV = _CHUNK // 32
_NV_LAST = _LASTC // 32
_NBINS = 11
_L = 16

_MAGIC = np.int32(0x5F3759DF)
_THR = [0.1 * i + 0.05 for i in range(10)]


def _rsqrt_newton(s):
    r = lax.bitcast_convert_type(
        _MAGIC - lax.shift_right_arithmetic(
            lax.bitcast_convert_type(s, jnp.int32), 1), jnp.float32)
    for _ in range(2):
        r = r * (1.5 - 0.5 * s * r * r)
    return r


def _sc_body(pr_hbm, tr_hbm, pdt_hbm, tdt_hbm, out_hbm,
             bufp, buft, bufpx, bufpy, buftx, bufty,
             hp, ht, hm, hd, row64, sem):
    cid = lax.axis_index("c")
    sid = lax.axis_index("s")
    im = cid * 2 + sid // 8
    sub = sid % 8
    lane = lax.iota(jnp.int32, _L)
    base = sub * _CHUNK

    def stage(n):
        cps = [
            pltpu.async_copy(pr_hbm.at[im, pl.ds(base, n)], bufp.at[pl.ds(0, n)], sem),
            pltpu.async_copy(tr_hbm.at[im, pl.ds(base, n)], buft.at[pl.ds(0, n)], sem),
            pltpu.async_copy(pdt_hbm.at[im, 0, pl.ds(base, n)], bufpx.at[pl.ds(0, n)], sem),
            pltpu.async_copy(pdt_hbm.at[im, 1, pl.ds(base, n)], bufpy.at[pl.ds(0, n)], sem),
            pltpu.async_copy(tdt_hbm.at[im, 0, pl.ds(base, n)], buftx.at[pl.ds(0, n)], sem),
            pltpu.async_copy(tdt_hbm.at[im, 1, pl.ds(base, n)], bufty.at[pl.ds(0, n)], sem),
        ]
        for cp in cps:
            cp.wait()

    @pl.when(sub == 7)
    def _():
        stage(_LASTC)

    @pl.when(sub != 7)
    def _():
        stage(_CHUNK)

    zero = jnp.zeros((_L,), jnp.float32)
    for b in range(_NBINS):
        hp[pl.ds(b * _L, _L)] = zero
        ht[pl.ds(b * _L, _L)] = zero
        hm[pl.ds(b * _L, _L)] = zero
        hd[pl.ds(b * _L, _L)] = zero

    nv = jnp.where(sub == 7, _NV_LAST, _NV)
    ones = jnp.ones((_L,), jnp.float32)

    def body(j, _):
        for u in range(2):
            o = j * (2 * _L) + u * _L
            p = bufp[pl.ds(o, _L)]
            t = buft[pl.ds(o, _L)]
            dx = buftx[pl.ds(o, _L)] - bufpx[pl.ds(o, _L)]
            dy = bufty[pl.ds(o, _L)] - bufpy[pl.ds(o, _L)]
            s = dx * dx + dy * dy
            dist = s * _rsqrt_newton(s)
            sig = 1.0 / (1.0 + jnp.exp(-p))
            lvl_p = ((sig + 0.05) * 10.0).astype(jnp.int32)
            lvl_t = ((t + 0.05) * 10.0).astype(jnp.int32)
            lvl_m = jnp.minimum(lvl_p, lvl_t)
            plsc.addupdate_scatter(hp, [lvl_p * _L + lane], ones)
            plsc.addupdate_scatter(ht, [lvl_t * _L + lane], ones)
            plsc.addupdate_scatter(hm, [lvl_m * _L + lane], ones)
            plsc.addupdate_scatter(hd, [lvl_m * _L + lane], dist)
        return _

    lax.fori_loop(0, nv, body, None)

    for slot, h in enumerate((hp, ht, hm, hd)):
        binvec = jnp.zeros((_L,), jnp.float32)
        for b in range(_NBINS):
            sb = jnp.sum(h[pl.ds(b * _L, _L)])
            binvec = jnp.where(lane == b, sb, binvec)
        row64[pl.ds(slot * _L, _L)] = binvec
    row = im * 8 + sub
    pltpu.sync_copy(row64, out_hbm.at[pl.ds(row * (4 * _L), 4 * _L)])


_sc_kernel = functools.partial(
    pl.kernel,
    mesh=plsc.VectorSubcoreMesh(core_axis_name="c", subcore_axis_name="s"),
    out_type=jax.ShapeDtypeStruct((32 * 4 * _L,), jnp.float32),
    compiler_params=pltpu.CompilerParams(needs_layout_passes=False),
    scratch_types=[
        pltpu.VMEM((_CHUNK,), jnp.float32),
        pltpu.VMEM((_CHUNK,), jnp.float32),
        pltpu.VMEM((_CHUNK,), jnp.float32),
        pltpu.VMEM((_CHUNK,), jnp.float32),
        pltpu.VMEM((_CHUNK,), jnp.float32),
        pltpu.VMEM((_CHUNK,), jnp.float32),
        pltpu.VMEM((_NBINS * _L,), jnp.float32),
        pltpu.VMEM((_NBINS * _L,), jnp.float32),
        pltpu.VMEM((_NBINS * _L,), jnp.float32),
        pltpu.VMEM((_NBINS * _L,), jnp.float32),
        pltpu.VMEM((4 * _L,), jnp.float32),
        pltpu.SemaphoreType.DMA,
    ],
)(_sc_body)


def _tc_body(s_ref, pr_ref, tr_ref, pdt_ref, tdt_ref, pr_out, rc_out, de_out):
    lane16 = lax.iota(jnp.int32, _L)

    pr_acc = jnp.float32(0.0)
    rc_acc = jnp.float32(0.0)
    de_acc = jnp.float32(0.0)
    for im in range(_N_IMG):
        rowsum = jnp.zeros((4 * _L,), jnp.float32)
        for w in range(8):
            rowsum = rowsum + s_ref[pl.ds((im * 8 + w) * 4 * _L, 4 * _L)]
        p = pr_ref[im]
        t = tr_ref[im]
        sig = jax.nn.sigmoid(p)
        dx = tdt_ref[im, 0] - pdt_ref[im, 0]
        dy = tdt_ref[im, 1] - pdt_ref[im, 1]
        dist = jnp.sqrt(dx * dx + dy * dy)
        sfx = []
        for slot in range(4):
            h = lax.slice(rowsum, (slot * _L,), ((slot + 1) * _L,))
            v = jnp.zeros((_L,), jnp.float32)
            for i in range(10):
                sc_part = jnp.sum(jnp.where(lane16 > i, h, 0.0))
                pos = sig > _THR[i]
                tru = t > _THR[i]
                if slot == 0:
                    cnt = jnp.sum(pos.astype(jnp.float32))
                elif slot == 1:
                    cnt = jnp.sum(tru.astype(jnp.float32))
                elif slot == 2:
                    cnt = jnp.sum(jnp.logical_and(pos, tru).astype(jnp.float32))
                else:
                    cnt = jnp.sum(jnp.where(jnp.logical_and(pos, tru), dist, 0.0))
                v = jnp.where(lane16 == i, sc_part + cnt, v)
            sfx.append(v)
        pos_s, tru_s, tp_s, d_s = sfx
        pr_acc = pr_acc + jnp.sum(tp_s / (pos_s + 1.0))
        rc_acc = rc_acc + jnp.sum(tp_s / (tru_s + 1.0))
        de_acc = de_acc + jnp.sum(d_s / jnp.maximum(tp_s, 1.0))
    scale = jnp.float32(1.0 / (10.0 * _N_IMG))
    pr_out[0, 0] = pr_acc * scale
    rc_out[0, 0] = rc_acc * scale
    de_out[0, 0] = de_acc * scale


def _tc_merge(s, pr_tail, tr_tail, pdt_tail, tdt_tail):
    return pl.pallas_call(
        _tc_body,
        out_shape=(
            jax.ShapeDtypeStruct((1, 1), jnp.float32),
            jax.ShapeDtypeStruct((1, 1), jnp.float32),
            jax.ShapeDtypeStruct((1, 1), jnp.float32),
        ),
        out_specs=(
            pl.BlockSpec(memory_space=pltpu.SMEM),
            pl.BlockSpec(memory_space=pltpu.SMEM),
            pl.BlockSpec(memory_space=pltpu.SMEM),
        ),
    )(s, pr_tail, tr_tail, pdt_tail, tdt_tail)


def kernel(pred_rank, pred_disp_vector, target_rank, target_disp_vector, anchors):
    del anchors
    pdt = jnp.transpose(pred_disp_vector, (0, 2, 1))
    tdt = jnp.transpose(target_disp_vector, (0, 2, 1))
    s = _sc_kernel(pred_rank, target_rank, pdt, tdt)
    o = _tc_merge(
        s,
        pred_rank[:, _CUT:],
        target_rank[:, _CUT:],
        pdt[:, :, _CUT:],
        tdt[:, :, _CUT:],
    )
    return (o[0][0, 0], o[1][0, 0], o[2][0, 0])
